# Initial kernel scaffold; baseline (speedup 1.0000x reference)
import jax
import jax.numpy as jnp
from jax.experimental import pallas as pl


def kernel(user_emb, items, items_emb, W1, b1, W2, b2, W3, b3):
    raise NotImplementedError("write your pallas kernel here")


# speedup vs baseline: 2.4926x; 2.4926x over previous
import functools

import jax
import jax.numpy as jnp
from jax import lax
from jax.experimental import pallas as pl
from jax.experimental.pallas import tpu as pltpu
from jax.experimental.pallas import tpu_sc as plsc

B, L, V, D = 16384, 20, 100000, 64
N = B * L
NC, NS = 2, 16
NW = NC * NS
ROWS_W = N // NW
GSZ = 128
NG = ROWS_W // GSZ
GPR = 8
NR = NG // GPR
CHUNK = GSZ * GPR


def _sc_gather(table, idx):
    mesh = plsc.VectorSubcoreMesh(core_axis_name="c", subcore_axis_name="s")

    @functools.partial(
        pl.kernel,
        mesh=mesh,
        out_type=jax.ShapeDtypeStruct((N // GSZ, GSZ, D), jnp.float32),
        scratch_types=[
            pltpu.VMEM((NG, GSZ), jnp.int32),
            pltpu.VMEM((GPR, GSZ, D), jnp.float32),
            pltpu.SemaphoreType.DMA,
        ],
        compiler_params=pltpu.CompilerParams(use_tc_tiling_on_sc=False),
    )
    def k(table_hbm, idx_hbm, out_hbm, idx_v, rows_v, sem):
        wid = lax.axis_index("s") * NC + lax.axis_index("c")
        pltpu.sync_copy(idx_hbm.at[wid], idx_v)
        out_base = wid * (ROWS_W // GSZ)

        def round_body(r, carry):
            copies = [
                pltpu.async_copy(
                    table_hbm.at[idx_v.at[r * GPR + g]], rows_v.at[g], sem
                )
                for g in range(GPR)
            ]
            for c in copies:
                c.wait()
            pltpu.sync_copy(rows_v, out_hbm.at[pl.ds(out_base + r * GPR, GPR)])
            return carry

        lax.fori_loop(0, NR, round_body, 0)

    return k(table, idx)


def _mlp(u, e_t, w1uT, w1eT, b1, w2T, b2, w3, b3):
    R = 1024
    NB = B // R

    def body(u_ref, e_ref, w1u_ref, w1e_ref, b1_ref, w2_ref, b2_ref,
             w3_ref, b3_ref, o_ref):
        h = jnp.dot(u_ref[...], w1u_ref[...], preferred_element_type=jnp.float32)
        h = h + jnp.dot(e_ref[0], w1e_ref[...], preferred_element_type=jnp.float32)
        h = jnp.maximum(h + b1_ref[...], 0.0)
        h = jnp.maximum(
            jnp.dot(h, w2_ref[...], preferred_element_type=jnp.float32)
            + b2_ref[...],
            0.0,
        )
        o = jnp.dot(h, w3_ref[...], preferred_element_type=jnp.float32)
        o_ref[0] = o + b3_ref[...]

    return pl.pallas_call(
        body,
        grid=(NB, L),
        in_specs=[
            pl.BlockSpec((R, D), lambda b, l: (b, 0)),
            pl.BlockSpec((1, R, D), lambda b, l: (l, b, 0)),
            pl.BlockSpec((D, 64), lambda b, l: (0, 0)),
            pl.BlockSpec((D, 64), lambda b, l: (0, 0)),
            pl.BlockSpec((1, 64), lambda b, l: (0, 0)),
            pl.BlockSpec((64, 32), lambda b, l: (0, 0)),
            pl.BlockSpec((1, 32), lambda b, l: (0, 0)),
            pl.BlockSpec((32, 1), lambda b, l: (0, 0)),
            pl.BlockSpec((1, 1), lambda b, l: (0, 0)),
        ],
        out_specs=pl.BlockSpec((1, R, 1), lambda b, l: (l, b, 0)),
        out_shape=jax.ShapeDtypeStruct((L, B, 1), jnp.float32),
        compiler_params=pltpu.CompilerParams(
            dimension_semantics=("parallel", "arbitrary")
        ),
    )(u, e_t, w1uT, w1eT, b1, w2T, b2, w3, b3)


def kernel(user_emb, items, items_emb, W1, b1, W2, b2, W3, b3):
    idx = items.T.reshape(NW, NG, GSZ)
    e = _sc_gather(items_emb, idx)
    e_t = e.reshape(L, B, D)
    w1uT = W1[:, :D].T
    w1eT = W1[:, D:].T
    out_t = _mlp(
        user_emb, e_t, w1uT, w1eT,
        b1.reshape(1, 64), W2.T, b2.reshape(1, 32),
        W3.reshape(32, 1), b3.reshape(1, 1),
    )
    return out_t.reshape(L, B).T


# speedup vs baseline: 3.8285x; 1.5359x over previous
import functools

import jax
import jax.numpy as jnp
from jax import lax
from jax.experimental import pallas as pl
from jax.experimental.pallas import tpu as pltpu
from jax.experimental.pallas import tpu_sc as plsc

B, L, V, D = 16384, 20, 100000, 64
N = B * L
NC, NS = 2, 16
NW = NC * NS
ROWS_W = N // NW
GSZ = 128
NG = ROWS_W // GSZ
GPR = 8
NR = NG // GPR


def _sc_gather(table, idx):
    mesh = plsc.VectorSubcoreMesh(core_axis_name="c", subcore_axis_name="s")

    @functools.partial(
        pl.kernel,
        mesh=mesh,
        out_type=jax.ShapeDtypeStruct((N // GSZ, GSZ, D), jnp.float32),
        scratch_types=[
            pltpu.VMEM((NG, GSZ), jnp.int32),
            pltpu.VMEM((GPR, GSZ, D), jnp.float32),
            pltpu.SemaphoreType.DMA,
        ],
        compiler_params=pltpu.CompilerParams(use_tc_tiling_on_sc=False),
    )
    def k(table_hbm, idx_hbm, out_hbm, idx_v, rows_v, sem):
        wid = lax.axis_index("s") * NC + lax.axis_index("c")
        pltpu.sync_copy(idx_hbm.at[wid], idx_v)
        out_base = wid * (ROWS_W // GSZ)

        def round_body(r, carry):
            copies = [
                pltpu.async_copy(
                    table_hbm.at[idx_v.at[r * GPR + g]], rows_v.at[g], sem
                )
                for g in range(GPR)
            ]
            for c in copies:
                c.wait()
            pltpu.sync_copy(rows_v, out_hbm.at[pl.ds(out_base + r * GPR, GPR)])
            return carry

        lax.fori_loop(0, NR, round_body, 0)

    return k(table, idx)


def _mlp(u2, e2, w1u2, w1e2, b1_2, w2_2, b2_2, w3_2, b3_2):
    R2 = 512
    NB = (B // 2) // R2

    def body(u_ref, e_ref, w1u_ref, w1e_ref, b1_ref, w2_ref, b2_ref,
             w3_ref, b3_ref, o_ref, uh_ref):
        l = pl.program_id(1)

        @pl.when(l == 0)
        def _():
            uh_ref[...] = jnp.dot(
                u_ref[...], w1u_ref[...], preferred_element_type=jnp.float32
            )
            o_ref[...] = jnp.zeros_like(o_ref)

        h = uh_ref[...] + jnp.dot(
            e_ref[0], w1e_ref[...], preferred_element_type=jnp.float32
        )
        h = jnp.maximum(h + b1_ref[...], 0.0)
        h = jnp.maximum(
            jnp.dot(h, w2_ref[...], preferred_element_type=jnp.float32)
            + b2_ref[...],
            0.0,
        )
        o2 = (
            jnp.dot(h, w3_ref[...], preferred_element_type=jnp.float32)
            + b3_ref[...]
        )
        rows2 = lax.broadcasted_iota(jnp.int32, (2, 128), 0)
        cols = lax.broadcasted_iota(jnp.int32, (2, 128), 1)
        sel = (cols == rows2 * 64 + l).astype(jnp.float32)
        o_ref[...] += jnp.dot(o2, sel, preferred_element_type=jnp.float32)

    return pl.pallas_call(
        body,
        grid=(NB, L),
        in_specs=[
            pl.BlockSpec((R2, 128), lambda b, l: (b, 0)),
            pl.BlockSpec((1, R2, 128), lambda b, l: (l, b, 0)),
            pl.BlockSpec((128, 128), lambda b, l: (0, 0)),
            pl.BlockSpec((128, 128), lambda b, l: (0, 0)),
            pl.BlockSpec((1, 128), lambda b, l: (0, 0)),
            pl.BlockSpec((128, 64), lambda b, l: (0, 0)),
            pl.BlockSpec((1, 64), lambda b, l: (0, 0)),
            pl.BlockSpec((64, 2), lambda b, l: (0, 0)),
            pl.BlockSpec((1, 2), lambda b, l: (0, 0)),
        ],
        out_specs=pl.BlockSpec((R2, 128), lambda b, l: (b, 0)),
        out_shape=jax.ShapeDtypeStruct((B // 2, 128), jnp.float32),
        scratch_shapes=[pltpu.VMEM((R2, 128), jnp.float32)],
        compiler_params=pltpu.CompilerParams(
            dimension_semantics=("parallel", "arbitrary")
        ),
    )(u2, e2, w1u2, w1e2, b1_2, w2_2, b2_2, w3_2, b3_2)


def _blockdiag2(m):
    z = jnp.zeros_like(m)
    return jnp.concatenate(
        [jnp.concatenate([m, z], axis=1), jnp.concatenate([z, m], axis=1)],
        axis=0,
    )


def kernel(user_emb, items, items_emb, W1, b1, W2, b2, W3, b3):
    idx = items.T.reshape(NW, NG, GSZ)
    e = _sc_gather(items_emb, idx)
    e2 = e.reshape(L, B // 2, 128)
    u2 = user_emb.reshape(B // 2, 128)
    w1u2 = _blockdiag2(W1[:, :D].T)
    w1e2 = _blockdiag2(W1[:, D:].T)
    b1_2 = jnp.tile(b1.reshape(1, 64), (1, 2))
    w2_2 = _blockdiag2(W2.T)
    b2_2 = jnp.tile(b2.reshape(1, 32), (1, 2))
    w3_2 = _blockdiag2(W3.reshape(32, 1))
    b3_2 = jnp.tile(b3.reshape(1, 1), (1, 2))
    out_v = _mlp(u2, e2, w1u2, w1e2, b1_2, w2_2, b2_2, w3_2, b3_2)
    return out_v.reshape(B, 64)[:, :L]


# speedup vs baseline: 7.2620x; 1.8968x over previous
import functools

import jax
import jax.numpy as jnp
from jax import lax
from jax.experimental import pallas as pl
from jax.experimental.pallas import tpu as pltpu
from jax.experimental.pallas import tpu_sc as plsc

B, L, V, D = 16384, 20, 100000, 64
N = B * L
NC, NS = 2, 16
NW = NC * NS
ROWS_W = N // NW
GSZ = 128
NG = ROWS_W // GSZ
GPR = 8
NR = NG // GPR


def _sc_gather(table, idx):
    mesh = plsc.VectorSubcoreMesh(core_axis_name="c", subcore_axis_name="s")

    @functools.partial(
        pl.kernel,
        mesh=mesh,
        out_type=jax.ShapeDtypeStruct((N // GSZ, GSZ, D), jnp.float32),
        scratch_types=[
            pltpu.VMEM((NG, GSZ), jnp.int32),
            pltpu.VMEM((GPR, GSZ, D), jnp.float32),
            pltpu.SemaphoreType.DMA,
        ],
        compiler_params=pltpu.CompilerParams(use_tc_tiling_on_sc=False),
    )
    def k(table_hbm, idx_hbm, out_hbm, idx_v, rows_v, sem):
        wid = lax.axis_index("s") * NC + lax.axis_index("c")
        pltpu.sync_copy(idx_hbm.at[wid], idx_v)
        out_base = wid * (ROWS_W // GSZ)

        def round_body(r, carry):
            copies = [
                pltpu.async_copy(
                    table_hbm.at[idx_v.at[r * GPR + g]], rows_v.at[g], sem
                )
                for g in range(GPR)
            ]
            for c in copies:
                c.wait()
            pltpu.sync_copy(rows_v, out_hbm.at[pl.ds(out_base + r * GPR, GPR)])
            return carry

        lax.fori_loop(0, NR, round_body, 0)

    return k(table, idx)


def _mlp(u2, e2, w1u2, w1e2, b1_2, w2_2, b2_2, w3_2, b3_2):
    R2 = 8192
    NB = (B // 2) // R2

    def body(u_ref, e_ref, w1u_ref, w1e_ref, b1_ref, w2_ref, b2_ref,
             w3_ref, b3_ref, o_ref, uh_ref):
        l = pl.program_id(1)

        @pl.when(l == 0)
        def _():
            uh_ref[...] = jnp.dot(
                u_ref[...], w1u_ref[...], preferred_element_type=jnp.float32,
                precision=lax.Precision.DEFAULT
            )
            o_ref[...] = jnp.zeros_like(o_ref)

        h = uh_ref[...] + jnp.dot(
            e_ref[0], w1e_ref[...], preferred_element_type=jnp.float32,
            precision=lax.Precision.DEFAULT
        )
        h = jnp.maximum(h + b1_ref[...], 0.0)
        h = jnp.maximum(
            jnp.dot(h, w2_ref[...], preferred_element_type=jnp.float32,
                    precision=lax.Precision.DEFAULT)
            + b2_ref[...],
            0.0,
        )
        o2 = (
            jnp.dot(h, w3_ref[...], preferred_element_type=jnp.float32,
                    precision=lax.Precision.DEFAULT)
            + b3_ref[...]
        )
        rows2 = lax.broadcasted_iota(jnp.int32, (2, 128), 0)
        cols = lax.broadcasted_iota(jnp.int32, (2, 128), 1)
        sel = (cols == rows2 * 64 + l).astype(jnp.float32)
        o_ref[...] += jnp.dot(o2, sel, preferred_element_type=jnp.float32)

    return pl.pallas_call(
        body,
        grid=(NB, L),
        in_specs=[
            pl.BlockSpec((R2, 128), lambda b, l: (b, 0)),
            pl.BlockSpec((1, R2, 128), lambda b, l: (l, b, 0)),
            pl.BlockSpec((128, 128), lambda b, l: (0, 0)),
            pl.BlockSpec((128, 128), lambda b, l: (0, 0)),
            pl.BlockSpec((1, 128), lambda b, l: (0, 0)),
            pl.BlockSpec((128, 64), lambda b, l: (0, 0)),
            pl.BlockSpec((1, 64), lambda b, l: (0, 0)),
            pl.BlockSpec((64, 2), lambda b, l: (0, 0)),
            pl.BlockSpec((1, 2), lambda b, l: (0, 0)),
        ],
        out_specs=pl.BlockSpec((R2, 128), lambda b, l: (b, 0)),
        out_shape=jax.ShapeDtypeStruct((B // 2, 128), jnp.float32),
        scratch_shapes=[pltpu.VMEM((R2, 128), jnp.float32)],
        compiler_params=pltpu.CompilerParams(
            dimension_semantics=("parallel", "arbitrary")
        ),
    )(u2, e2, w1u2, w1e2, b1_2, w2_2, b2_2, w3_2, b3_2)


def _blockdiag2(m):
    z = jnp.zeros_like(m)
    return jnp.concatenate(
        [jnp.concatenate([m, z], axis=1), jnp.concatenate([z, m], axis=1)],
        axis=0,
    )


def kernel(user_emb, items, items_emb, W1, b1, W2, b2, W3, b3):
    idx = items.T.reshape(NW, NG, GSZ)
    e = _sc_gather(items_emb, idx)
    e2 = e.reshape(L, B // 2, 128)
    u2 = user_emb.reshape(B // 2, 128)
    w1u2 = _blockdiag2(W1[:, :D].T)
    w1e2 = _blockdiag2(W1[:, D:].T)
    b1_2 = jnp.tile(b1.reshape(1, 64), (1, 2))
    w2_2 = _blockdiag2(W2.T)
    b2_2 = jnp.tile(b2.reshape(1, 32), (1, 2))
    w3_2 = _blockdiag2(W3.reshape(32, 1))
    b3_2 = jnp.tile(b3.reshape(1, 1), (1, 2))
    out_v = _mlp(u2, e2, w1u2, w1e2, b1_2, w2_2, b2_2, w3_2, b3_2)
    return out_v.reshape(B, 64)[:, :L]


# speedup vs baseline: 7.6126x; 1.0483x over previous
import functools

import jax
import jax.numpy as jnp
from jax import lax
from jax.experimental import pallas as pl
from jax.experimental.pallas import tpu as pltpu
from jax.experimental.pallas import tpu_sc as plsc

B, L, V, D = 16384, 20, 100000, 64
NC, NS = 2, 16
NW = NC * NS
GSZ = 128
GPR = 8
CH = 2
BC = B // CH


def _sc_gather(table, idx, n):
    rows_w = n // NW
    ng = rows_w // GSZ
    nr = ng // GPR
    mesh = plsc.VectorSubcoreMesh(core_axis_name="c", subcore_axis_name="s")

    @functools.partial(
        pl.kernel,
        mesh=mesh,
        out_type=jax.ShapeDtypeStruct((n // GSZ, GSZ, D), jnp.float32),
        scratch_types=[
            pltpu.VMEM((ng, GSZ), jnp.int32),
            pltpu.VMEM((GPR, GSZ, D), jnp.float32),
            pltpu.SemaphoreType.DMA,
        ],
        compiler_params=pltpu.CompilerParams(use_tc_tiling_on_sc=False),
    )
    def k(table_hbm, idx_hbm, out_hbm, idx_v, rows_v, sem):
        wid = lax.axis_index("s") * NC + lax.axis_index("c")
        pltpu.sync_copy(idx_hbm.at[wid], idx_v)
        out_base = wid * (rows_w // GSZ)

        def round_body(r, carry):
            copies = [
                pltpu.async_copy(
                    table_hbm.at[idx_v.at[r * GPR + g]], rows_v.at[g], sem
                )
                for g in range(GPR)
            ]
            for c in copies:
                c.wait()
            pltpu.sync_copy(rows_v, out_hbm.at[pl.ds(out_base + r * GPR, GPR)])
            return carry

        lax.fori_loop(0, nr, round_body, 0)

    return k(table, idx)


def _mlp(u2, e2, w1u2, w1e2, b1_2, w2_2, b2_2, w3_2, b3_2):
    R2 = BC // 2

    def body(u_ref, e_ref, w1u_ref, w1e_ref, b1_ref, w2_ref, b2_ref,
             w3_ref, b3_ref, o_ref, uh_ref):
        l = pl.program_id(0)

        @pl.when(l == 0)
        def _():
            uh_ref[...] = jnp.dot(
                u_ref[...], w1u_ref[...], preferred_element_type=jnp.float32
            )
            o_ref[...] = jnp.zeros_like(o_ref)

        h = uh_ref[...] + jnp.dot(
            e_ref[0], w1e_ref[...], preferred_element_type=jnp.float32
        )
        h = jnp.maximum(h + b1_ref[...], 0.0)
        h = jnp.maximum(
            jnp.dot(h, w2_ref[...], preferred_element_type=jnp.float32)
            + b2_ref[...],
            0.0,
        )
        o2 = (
            jnp.dot(h, w3_ref[...], preferred_element_type=jnp.float32)
            + b3_ref[...]
        )
        rows2 = lax.broadcasted_iota(jnp.int32, (2, 128), 0)
        cols = lax.broadcasted_iota(jnp.int32, (2, 128), 1)
        sel = (cols == rows2 * 64 + l).astype(jnp.float32)
        o_ref[...] += jnp.dot(o2, sel, preferred_element_type=jnp.float32)

    return pl.pallas_call(
        body,
        grid=(L,),
        in_specs=[
            pl.BlockSpec((R2, 128), lambda l: (0, 0)),
            pl.BlockSpec((1, R2, 128), lambda l: (l, 0, 0)),
            pl.BlockSpec((128, 128), lambda l: (0, 0)),
            pl.BlockSpec((128, 128), lambda l: (0, 0)),
            pl.BlockSpec((1, 128), lambda l: (0, 0)),
            pl.BlockSpec((128, 64), lambda l: (0, 0)),
            pl.BlockSpec((1, 64), lambda l: (0, 0)),
            pl.BlockSpec((64, 2), lambda l: (0, 0)),
            pl.BlockSpec((1, 2), lambda l: (0, 0)),
        ],
        out_specs=pl.BlockSpec((R2, 128), lambda l: (0, 0)),
        out_shape=jax.ShapeDtypeStruct((BC // 2, 128), jnp.float32),
        scratch_shapes=[pltpu.VMEM((R2, 128), jnp.float32)],
        compiler_params=pltpu.CompilerParams(
            dimension_semantics=("arbitrary",)
        ),
    )(u2, e2, w1u2, w1e2, b1_2, w2_2, b2_2, w3_2, b3_2)


def _blockdiag2(m):
    z = jnp.zeros_like(m)
    return jnp.concatenate(
        [jnp.concatenate([m, z], axis=1), jnp.concatenate([z, m], axis=1)],
        axis=0,
    )


def kernel(user_emb, items, items_emb, W1, b1, W2, b2, W3, b3):
    t2 = lax.optimization_barrier(items_emb.reshape(V // 2, 128))
    table = t2.reshape(V, D)

    w1u2 = _blockdiag2(W1[:, :D].T)
    w1e2 = _blockdiag2(W1[:, D:].T)
    b1_2 = jnp.tile(b1.reshape(1, 64), (1, 2))
    w2_2 = _blockdiag2(W2.T)
    b2_2 = jnp.tile(b2.reshape(1, 32), (1, 2))
    w3_2 = _blockdiag2(W3.reshape(32, 1))
    b3_2 = jnp.tile(b3.reshape(1, 1), (1, 2))

    nc = BC * L
    ng = nc // NW // GSZ
    es = []
    for c in range(CH):
        idx_c = items[c * BC:(c + 1) * BC, :].T.reshape(NW, ng, GSZ)
        es.append(_sc_gather(table, idx_c, nc))
    outs = []
    for c in range(CH):
        e2 = es[c].reshape(L, BC // 2, 128)
        u2 = user_emb[c * BC:(c + 1) * BC, :].reshape(BC // 2, 128)
        outs.append(
            _mlp(u2, e2, w1u2, w1e2, b1_2, w2_2, b2_2, w3_2, b3_2)
        )
    out_v = jnp.concatenate(outs, axis=0)
    return out_v.reshape(B, 64)[:, :L]
